# baseline (speedup 1.0000x reference)
import functools

import jax
import jax.numpy as jnp
from jax import lax
from jax.experimental import pallas as pl
from jax.experimental.pallas import tpu as pltpu
from jax.experimental.pallas import tpu_sc as plsc

N_NODE = 10000
E_TOTAL = 320000
B_Q = 512
D = 128
DX = 256
A = 32
C = 32
NW = 32
NCHUNK = E_TOTAL // C
CH_MAIN = 312
QCH = 26
QE = QCH * C
E_MAIN = CH_MAIN * C
N_EPI = NCHUNK - CH_MAIN * NW
ROWS_A = 640


def _mm_block(x_ref, w_ref, o_ref):
    o_ref[...] = jnp.dot(x_ref[...], w_ref[...], preferred_element_type=jnp.float32)


def _mm(x, w, block_rows=2000):
    n, d = x.shape
    k = w.shape[1]
    grid = pl.cdiv(n, block_rows)
    return pl.pallas_call(
        _mm_block,
        grid=(grid,),
        in_specs=[
            pl.BlockSpec((block_rows, d), lambda i: (i, 0)),
            pl.BlockSpec((d, k), lambda i: (0, 0)),
        ],
        out_specs=pl.BlockSpec((block_rows, k), lambda i: (i, 0)),
        out_shape=jax.ShapeDtypeStruct((n, k), jnp.float32),
    )(x, w)


def _post_block(p0_ref, p1_ref, w_ref, o_ref):
    s = p0_ref[...] + p1_ref[...]
    o_ref[...] = jnp.dot(s, w_ref[...], preferred_element_type=jnp.float32)


def _post(partials, wh, block_rows=2000):
    n = partials.shape[0] // 2
    grid = n // block_rows
    off = n // block_rows
    return pl.pallas_call(
        _post_block,
        grid=(grid,),
        in_specs=[
            pl.BlockSpec((block_rows, D), lambda i: (i, 0)),
            pl.BlockSpec((block_rows, D), lambda i, _o=off: (i + _o, 0)),
            pl.BlockSpec((D, D), lambda i: (0, 0)),
        ],
        out_specs=pl.BlockSpec((block_rows, D), lambda i: (i, 0)),
        out_shape=jax.ShapeDtypeStruct((n, D), jnp.float32),
    )(partials, partials, wh)


def _sc_body(sub_h, rel_h, obj_h, r_idx_h, q_rel_h, hx_h, rx_h, wqr_h, sp_h,
             out_h,
             subB, relB, objB, ridxB, qidx_v, obj_u, hsx0, hrx0, hsx1, hrx1,
             mq, mm, alpha_v, spv, qrl, acc, gsem0, gsem1, qsem, ssem):
    cid = lax.axis_index("c")
    sid = lax.axis_index("s")
    wid = sid * 2 + cid
    ebase = wid * E_MAIN

    pltpu.sync_copy(sp_h, spv)
    pltpu.sync_copy(q_rel_h, qrl.at[pl.ds(0, B_Q)])
    b0 = spv[pl.ds(32, 16)]
    b1 = spv[pl.ds(48, 16)]
    wv0 = spv[pl.ds(0, 16)]
    wv1 = spv[pl.ds(16, 16)]
    wb = spv[pl.ds(64, 16)][0]

    z16 = jnp.zeros((16,), jnp.float32)

    def _zrow(r, carry):
        for k in range(D // 16):
            mm[r, pl.ds(k * 16, 16)] = z16
        return carry

    lax.fori_loop(0, C, _zrow, 0)
    z0 = sid * (N_NODE // 16)
    for j in range(18):
        pltpu.sync_copy(mm, acc.at[pl.ds(z0 + j * C, C), :])
    pltpu.sync_copy(mm.at[pl.ds(0, 17), :], acc.at[pl.ds(z0 + 19 * C, 17), :])
    pltpu.async_copy(mm, acc.at[pl.ds(z0 + 18 * C, C), :], ssem)
    plsc.subcore_barrier()

    lane = lax.iota(jnp.int32, 16)
    nm1 = jnp.full((16,), N_NODE - 1, jnp.int32)
    perms = [lane ^ k for k in (1, 2, 4, 8)]
    hbufs = (hsx0, hsx1)
    rbufs = (hrx0, hrx1)
    gsems = (gsem0, gsem1)

    def _compose(kq):
        off = kq * C

        @plsc.parallel_loop(0, C // 16, 1, unroll=2)
        def _cg(g):
            r16 = ridxB[pl.ds(off + g * 16, 16)]
            qv = jnp.zeros((16,), jnp.int32)
            for j in range(16):
                qv = jnp.where(lane == j, qrl[pl.ds(r16[j], 16)][0], qv)
            qidx_v[pl.ds(g * 16, 16)] = qv

    def _fire_rows(kq, s):
        off = kq * C
        pltpu.async_copy(hx_h.at[subB.at[pl.ds(off, C)]], hbufs[s], gsems[s])
        pltpu.async_copy(rx_h.at[relB.at[pl.ds(off, C)]], rbufs[s], gsems[s])

    def _fire_wqr(kq):
        _compose(kq)
        pltpu.async_copy(wqr_h.at[qidx_v], mq, qsem)

    def _wait_rows(s):
        pltpu.make_async_copy(hx_h.at[subB.at[pl.ds(0, C)]], hbufs[s],
                              gsems[s]).wait()
        pltpu.make_async_copy(rx_h.at[relB.at[pl.ds(0, C)]], rbufs[s],
                              gsems[s]).wait()

    def _wait_wqr():
        pltpu.make_async_copy(wqr_h.at[qidx_v], mq, qsem).wait()

    def _process(kq, s, fire_next):
        hx = hbufs[s]
        rx = rbufs[s]
        _wait_rows(s)
        _wait_wqr()
        pltpu.make_async_copy(mm, acc.at[obj_u], ssem).wait()
        off = kq * C
        for g in range(C // 16):
            obj_u[pl.ds(g * 16, 16)] = jnp.minimum(
                objB[pl.ds(off + g * 16, 16)], nm1)
        @plsc.parallel_loop(0, C, 1, unroll=4)
        def _edge(e):
            v0 = jnp.maximum(hx[e, pl.ds(D, 16)] + rx[e, pl.ds(D, 16)]
                             + mq[e, pl.ds(0, 16)] + b0, 0.0) * wv0
            v1 = jnp.maximum(hx[e, pl.ds(D + 16, 16)]
                             + rx[e, pl.ds(D + 16, 16)]
                             + mq[e, pl.ds(16, 16)] + b1, 0.0) * wv1
            s = v0 + v1
            for p in perms:
                s = s + s.at[p].get(mode="promise_in_bounds")
            a = 1.0 / (1.0 + jnp.exp(-(s + wb)))
            for kk in range(D // 16):
                sl = pl.ds(kk * 16, 16)
                mm[e, sl] = hx[e, sl] * rx[e, sl] * a

        if fire_next is not None:
            kn, cond = fire_next
            if cond is None:
                _fire_wqr(kn)
            else:
                @pl.when(cond)
                def _():
                    _fire_wqr(kn)
        pltpu.async_copy(mm, acc.at[obj_u], ssem, add=True)

    def _quarter(q, carry):
        qbase = ebase + q * QE
        pltpu.sync_copy(sub_h.at[pl.ds(qbase, QE)], subB)
        pltpu.sync_copy(rel_h.at[pl.ds(qbase, QE)], relB)
        pltpu.sync_copy(obj_h.at[pl.ds(qbase, QE)], objB)
        pltpu.sync_copy(r_idx_h.at[pl.ds(qbase, QE)], ridxB)
        _fire_rows(0, 0)
        _fire_wqr(0)

        def _pair(t, pc):
            k0 = t * 2
            _fire_rows(k0 + 1, 1)
            _process(k0, 0, (k0 + 1, None))

            @pl.when(t < QCH // 2 - 1)
            def _():
                _fire_rows(k0 + 2, 0)
            _process(k0 + 1, 1, (k0 + 2, t < QCH // 2 - 1))
            return pc

        lax.fori_loop(0, QCH // 2, _pair, 0)
        return carry

    lax.fori_loop(0, 12, _quarter, 0)

    @pl.when(wid < N_EPI)
    def _():
        base = NW * E_MAIN + wid * C
        pltpu.sync_copy(sub_h.at[pl.ds(base, C)], subB.at[pl.ds(0, C)])
        pltpu.sync_copy(rel_h.at[pl.ds(base, C)], relB.at[pl.ds(0, C)])
        pltpu.sync_copy(obj_h.at[pl.ds(base, C)], objB.at[pl.ds(0, C)])
        pltpu.sync_copy(r_idx_h.at[pl.ds(base, C)], ridxB.at[pl.ds(0, C)])
        _fire_rows(0, 0)
        _fire_wqr(0)
        _process(0, 0, None)

    pltpu.make_async_copy(mm, acc.at[obj_u], ssem).wait()
    plsc.subcore_barrier()
    o0 = cid * N_NODE + sid * ROWS_A

    @pl.when(sid < 15)
    def _():
        pltpu.sync_copy(acc.at[pl.ds(sid * ROWS_A, ROWS_A), :],
                        out_h.at[pl.ds(o0, ROWS_A), :])

    @pl.when(sid == 15)
    def _():
        pltpu.sync_copy(acc.at[pl.ds(15 * ROWS_A, N_NODE - 15 * ROWS_A), :],
                        out_h.at[pl.ds(o0, N_NODE - 15 * ROWS_A), :])


@functools.cache
def _sc_edges_fn():
  return pl.kernel(
    _sc_body,
    out_type=jax.ShapeDtypeStruct((2 * N_NODE, D), jnp.float32),
    mesh=plsc.VectorSubcoreMesh(core_axis_name="c", subcore_axis_name="s",
                                num_cores=2, num_subcores=16),
    compiler_params=pltpu.CompilerParams(needs_layout_passes=False),
    scratch_types=[
        pltpu.VMEM((QE,), jnp.int32),
        pltpu.VMEM((QE,), jnp.int32),
        pltpu.VMEM((QE,), jnp.int32),
        pltpu.VMEM((QE,), jnp.int32),
        pltpu.VMEM((C,), jnp.int32),
        pltpu.VMEM((C,), jnp.int32),
        pltpu.VMEM((C, DX), jnp.float32),
        pltpu.VMEM((C, DX), jnp.float32),
        pltpu.VMEM((C, DX), jnp.float32),
        pltpu.VMEM((C, DX), jnp.float32),
        pltpu.VMEM((C, D), jnp.float32),
        pltpu.VMEM((C, D), jnp.float32),
        pltpu.VMEM((C,), jnp.float32),
        pltpu.VMEM((80,), jnp.float32),
        pltpu.VMEM((B_Q + 16,), jnp.int32),
        pltpu.VMEM_SHARED((N_NODE, D), jnp.float32),
        pltpu.SemaphoreType.DMA,
        pltpu.SemaphoreType.DMA,
        pltpu.SemaphoreType.DMA,
        pltpu.SemaphoreType.DMA,
    ],
  )


def kernel(q_sub, q_rel, r_idx, hidden, edges, n_node, rela_embed, Ws, Wr,
           Wqr_W, Wqr_b, walpha_W, walpha_b, Wh):
    sub_e = edges[:, 0].astype(jnp.int32)
    rel_e = edges[:, 1].astype(jnp.int32)
    obj_e = edges[:, 2].astype(jnp.int32)
    a_sub = _mm(hidden, Ws)
    a_rel = _mm(rela_embed, Wr)
    wqr_pre = _mm(rela_embed, Wqr_W)
    hx = jnp.concatenate(
        [hidden, a_sub, jnp.zeros((N_NODE, DX - D - A), jnp.float32)], axis=1)
    rx = jnp.concatenate(
        [rela_embed, a_rel,
         jnp.zeros((rela_embed.shape[0], DX - D - A), jnp.float32)], axis=1)
    wqrx = jnp.concatenate(
        [wqr_pre, jnp.zeros((wqr_pre.shape[0], D - A), jnp.float32)], axis=1)
    sp = jnp.concatenate([
        walpha_W.reshape(-1), Wqr_b.reshape(-1), walpha_b.reshape(-1),
        jnp.zeros((80 - A - A - 1,), jnp.float32),
    ])
    partials = _sc_edges_fn()(sub_e, rel_e, obj_e, r_idx.astype(jnp.int32),
                              q_rel.astype(jnp.int32), hx, rx, wqrx, sp)
    return _post(partials, Wh)[:N_NODE]


# speedup vs baseline: 1.0278x; 1.0278x over previous
import functools

import jax
import jax.numpy as jnp
from jax import lax
from jax.experimental import pallas as pl
from jax.experimental.pallas import tpu as pltpu
from jax.experimental.pallas import tpu_sc as plsc

N_NODE = 10000
E_TOTAL = 320000
B_Q = 512
D = 128
DX = 256
A = 32
C = 32
NW = 32
NCHUNK = E_TOTAL // C
CH_MAIN = 312
QCH = 26
QE = QCH * C
E_MAIN = CH_MAIN * C
N_EPI = NCHUNK - CH_MAIN * NW
ROWS_A = 640


def _mm_block(x_ref, w_ref, o_ref):
    o_ref[...] = jnp.dot(x_ref[...], w_ref[...], preferred_element_type=jnp.float32)


def _mm(x, w, block_rows=2000):
    n, d = x.shape
    k = w.shape[1]
    grid = pl.cdiv(n, block_rows)
    return pl.pallas_call(
        _mm_block,
        grid=(grid,),
        in_specs=[
            pl.BlockSpec((block_rows, d), lambda i: (i, 0)),
            pl.BlockSpec((d, k), lambda i: (0, 0)),
        ],
        out_specs=pl.BlockSpec((block_rows, k), lambda i: (i, 0)),
        out_shape=jax.ShapeDtypeStruct((n, k), jnp.float32),
    )(x, w)


def _post_block(p0_ref, p1_ref, w_ref, o_ref):
    s = p0_ref[...] + p1_ref[...]
    o_ref[...] = jnp.dot(s, w_ref[...], preferred_element_type=jnp.float32)


def _post(partials, wh, block_rows=2000):
    n = partials.shape[0] // 2
    grid = n // block_rows
    off = n // block_rows
    return pl.pallas_call(
        _post_block,
        grid=(grid,),
        in_specs=[
            pl.BlockSpec((block_rows, D), lambda i: (i, 0)),
            pl.BlockSpec((block_rows, D), lambda i, _o=off: (i + _o, 0)),
            pl.BlockSpec((D, D), lambda i: (0, 0)),
        ],
        out_specs=pl.BlockSpec((block_rows, D), lambda i: (i, 0)),
        out_shape=jax.ShapeDtypeStruct((n, D), jnp.float32),
    )(partials, partials, wh)


def _sc_body(sub_h, rel_h, obj_h, r_idx_h, q_rel_h, hx_h, wqr_h, sp_h,
             out_h,
             subB, relB, objB, ridxB, qidx_v, obj_u, hsx0, hrx0, hsx1, hrx1,
             mq, mm, alpha_v, spv, qrl, acc, gsem0, gsem1, qsem):
    del alpha_v
    cid = lax.axis_index("c")
    sid = lax.axis_index("s")
    wid = sid * 2 + cid
    ebase = wid * E_MAIN

    pltpu.sync_copy(sp_h, spv)
    pltpu.sync_copy(q_rel_h, qrl.at[pl.ds(0, B_Q)])
    b0 = spv[pl.ds(32, 16)]
    b1 = spv[pl.ds(48, 16)]
    wv0 = spv[pl.ds(0, 16)]
    wv1 = spv[pl.ds(16, 16)]
    wb = spv[pl.ds(64, 16)][0]

    z16 = jnp.zeros((16,), jnp.float32)

    def _zrow(r, carry):
        for k in range(D // 16):
            mm[r, pl.ds(k * 16, 16)] = z16
        return carry

    lax.fori_loop(0, C, _zrow, 0)
    z0 = sid * (N_NODE // 16)
    for j in range(19):
        pltpu.sync_copy(mm, acc.at[pl.ds(z0 + j * C, C), :])
    pltpu.sync_copy(mm.at[pl.ds(0, 17), :], acc.at[pl.ds(z0 + 19 * C, 17), :])
    plsc.subcore_barrier()

    lane = lax.iota(jnp.int32, 16)
    nm1 = jnp.full((16,), N_NODE - 1, jnp.int32)
    perms = [lane ^ k for k in (1, 2, 4, 8)]
    hbufs = (hsx0, hsx1)
    cidxs = (hrx0, hrx1)
    gsems = (gsem0, gsem1)

    def _compose(kq):
        off = kq * C

        @plsc.parallel_loop(0, C // 16, 1, unroll=2)
        def _cg(g):
            r16 = ridxB[pl.ds(off + g * 16, 16)]
            qv = jnp.zeros((16,), jnp.int32)
            for j in range(16):
                qv = jnp.where(lane == j, qrl[pl.ds(r16[j], 16)][0], qv)
            qidx_v[pl.ds(g * 16, 16)] = qv

    def _fire_rows(kq, s):
        _make_cidx(kq, s)
        pltpu.async_copy(hx_h.at[cidxs[s]], hbufs[s], gsems[s])

    def _fire_wqr(kq):
        _compose(kq)
        pltpu.async_copy(wqr_h.at[qidx_v], mq, qsem)

    def _wait_rows(s):
        pltpu.make_async_copy(hx_h.at[cidxs[s]], hbufs[s], gsems[s]).wait()

    def _make_cidx(kq, s):
        off = kq * C
        nn = jnp.full((16,), N_NODE, jnp.int32)
        for g in range(C // 16):
            cidxs[s][pl.ds(g * 16, 16)] = subB[pl.ds(off + g * 16, 16)]
            cidxs[s][pl.ds(C + g * 16, 16)] = relB[pl.ds(off + g * 16, 16)] + nn

    def _wait_wqr():
        pltpu.make_async_copy(wqr_h.at[qidx_v], mq, qsem).wait()

    def _process(kq, s, fire_next):
        hx = hbufs[s]
        _wait_rows(s)
        _wait_wqr()
        off = kq * C
        for g in range(C // 16):
            obj_u[pl.ds(g * 16, 16)] = jnp.minimum(
                objB[pl.ds(off + g * 16, 16)], nm1)
        @plsc.parallel_loop(0, C, 1, unroll=4)
        def _edge(e):
            v0 = jnp.maximum(hx[e, pl.ds(D, 16)] + hx[C + e, pl.ds(D, 16)]
                             + mq[e, pl.ds(0, 16)] + b0, 0.0) * wv0
            v1 = jnp.maximum(hx[e, pl.ds(D + 16, 16)]
                             + hx[C + e, pl.ds(D + 16, 16)]
                             + mq[e, pl.ds(16, 16)] + b1, 0.0) * wv1
            s = v0 + v1
            for p in perms:
                s = s + s.at[p].get(mode="promise_in_bounds")
            a = 1.0 / (1.0 + jnp.exp(-(s + wb)))
            for kk in range(D // 16):
                sl = pl.ds(kk * 16, 16)
                mm[e, sl] = hx[e, sl] * hx[C + e, sl] * a

        if fire_next is not None:
            kn, cond = fire_next
            if cond is None:
                _fire_wqr(kn)
            else:
                @pl.when(cond)
                def _():
                    _fire_wqr(kn)
        pltpu.sync_copy(mm, acc.at[obj_u], add=True)

    def _quarter(q, carry):
        qbase = ebase + q * QE
        pltpu.sync_copy(sub_h.at[pl.ds(qbase, QE)], subB)
        pltpu.sync_copy(rel_h.at[pl.ds(qbase, QE)], relB)
        pltpu.sync_copy(obj_h.at[pl.ds(qbase, QE)], objB)
        pltpu.sync_copy(r_idx_h.at[pl.ds(qbase, QE)], ridxB)
        _fire_rows(0, 0)
        _fire_wqr(0)

        def _pair(t, pc):
            k0 = t * 2
            _fire_rows(k0 + 1, 1)
            _process(k0, 0, (k0 + 1, None))

            @pl.when(t < QCH // 2 - 1)
            def _():
                _fire_rows(k0 + 2, 0)
            _process(k0 + 1, 1, (k0 + 2, t < QCH // 2 - 1))
            return pc

        lax.fori_loop(0, QCH // 2, _pair, 0)
        return carry

    lax.fori_loop(0, 12, _quarter, 0)

    @pl.when(wid < N_EPI)
    def _():
        base = NW * E_MAIN + wid * C
        pltpu.sync_copy(sub_h.at[pl.ds(base, C)], subB.at[pl.ds(0, C)])
        pltpu.sync_copy(rel_h.at[pl.ds(base, C)], relB.at[pl.ds(0, C)])
        pltpu.sync_copy(obj_h.at[pl.ds(base, C)], objB.at[pl.ds(0, C)])
        pltpu.sync_copy(r_idx_h.at[pl.ds(base, C)], ridxB.at[pl.ds(0, C)])
        _fire_rows(0, 0)
        _fire_wqr(0)
        _process(0, 0, None)

    plsc.subcore_barrier()
    o0 = cid * N_NODE + sid * ROWS_A

    @pl.when(sid < 15)
    def _():
        pltpu.sync_copy(acc.at[pl.ds(sid * ROWS_A, ROWS_A), :],
                        out_h.at[pl.ds(o0, ROWS_A), :])

    @pl.when(sid == 15)
    def _():
        pltpu.sync_copy(acc.at[pl.ds(15 * ROWS_A, N_NODE - 15 * ROWS_A), :],
                        out_h.at[pl.ds(o0, N_NODE - 15 * ROWS_A), :])


@functools.cache
def _sc_edges_fn():
  return pl.kernel(
    _sc_body,
    out_type=jax.ShapeDtypeStruct((2 * N_NODE, D), jnp.float32),
    mesh=plsc.VectorSubcoreMesh(core_axis_name="c", subcore_axis_name="s",
                                num_cores=2, num_subcores=16),
    compiler_params=pltpu.CompilerParams(needs_layout_passes=False),
    scratch_types=[
        pltpu.VMEM((QE,), jnp.int32),
        pltpu.VMEM((QE,), jnp.int32),
        pltpu.VMEM((QE,), jnp.int32),
        pltpu.VMEM((QE,), jnp.int32),
        pltpu.VMEM((C,), jnp.int32),
        pltpu.VMEM((C,), jnp.int32),
        pltpu.VMEM((2 * C, DX), jnp.float32),
        pltpu.VMEM((2 * C,), jnp.int32),
        pltpu.VMEM((2 * C, DX), jnp.float32),
        pltpu.VMEM((2 * C,), jnp.int32),
        pltpu.VMEM((C, D), jnp.float32),
        pltpu.VMEM((C, D), jnp.float32),
        pltpu.VMEM((C,), jnp.float32),
        pltpu.VMEM((80,), jnp.float32),
        pltpu.VMEM((B_Q + 16,), jnp.int32),
        pltpu.VMEM_SHARED((N_NODE, D), jnp.float32),
        pltpu.SemaphoreType.DMA,
        pltpu.SemaphoreType.DMA,
        pltpu.SemaphoreType.DMA,
    ],
  )


def kernel(q_sub, q_rel, r_idx, hidden, edges, n_node, rela_embed, Ws, Wr,
           Wqr_W, Wqr_b, walpha_W, walpha_b, Wh):
    sub_e = edges[:, 0].astype(jnp.int32)
    rel_e = edges[:, 1].astype(jnp.int32)
    obj_e = edges[:, 2].astype(jnp.int32)
    a_sub = _mm(hidden, Ws)
    a_rel = _mm(rela_embed, Wr)
    wqr_pre = _mm(rela_embed, Wqr_W)
    hx = jnp.concatenate(
        [hidden, a_sub, jnp.zeros((N_NODE, DX - D - A), jnp.float32)], axis=1)
    rx = jnp.concatenate(
        [rela_embed, a_rel,
         jnp.zeros((rela_embed.shape[0], DX - D - A), jnp.float32)], axis=1)
    wqrx = jnp.concatenate(
        [wqr_pre, jnp.zeros((wqr_pre.shape[0], D - A), jnp.float32)], axis=1)
    sp = jnp.concatenate([
        walpha_W.reshape(-1), Wqr_b.reshape(-1), walpha_b.reshape(-1),
        jnp.zeros((80 - A - A - 1,), jnp.float32),
    ])
    hxrx = jnp.concatenate([hx, rx], axis=0)
    partials = _sc_edges_fn()(sub_e, rel_e, obj_e, r_idx.astype(jnp.int32),
                              q_rel.astype(jnp.int32), hxrx, wqrx, sp)
    return _post(partials, Wh)[:N_NODE]


# speedup vs baseline: 1.1782x; 1.1463x over previous
import functools

import jax
import jax.numpy as jnp
from jax import lax
from jax.experimental import pallas as pl
from jax.experimental.pallas import tpu as pltpu
from jax.experimental.pallas import tpu_sc as plsc

N_NODE = 10000
E_TOTAL = 320000
B_Q = 512
D = 128
DX = 256
A = 32
C = 32
NW = 32
NCHUNK = E_TOTAL // C
CH_MAIN = 312
QCH = 26
QE = QCH * C
E_MAIN = CH_MAIN * C
N_EPI = NCHUNK - CH_MAIN * NW
ROWS_A = 640


def _mm_block(x_ref, w_ref, o_ref):
    o_ref[...] = jnp.dot(x_ref[...], w_ref[...], preferred_element_type=jnp.float32)


def _mm(x, w, block_rows=2000):
    n, d = x.shape
    k = w.shape[1]
    grid = pl.cdiv(n, block_rows)
    return pl.pallas_call(
        _mm_block,
        grid=(grid,),
        in_specs=[
            pl.BlockSpec((block_rows, d), lambda i: (i, 0)),
            pl.BlockSpec((d, k), lambda i: (0, 0)),
        ],
        out_specs=pl.BlockSpec((block_rows, k), lambda i: (i, 0)),
        out_shape=jax.ShapeDtypeStruct((n, k), jnp.float32),
    )(x, w)


def _post_block(p0_ref, p1_ref, w_ref, o_ref):
    s = p0_ref[...] + p1_ref[...]
    o_ref[...] = jnp.dot(s, w_ref[...], preferred_element_type=jnp.float32)


def _post(partials, wh, block_rows=2000):
    n = partials.shape[0] // 2
    grid = n // block_rows
    off = n // block_rows
    return pl.pallas_call(
        _post_block,
        grid=(grid,),
        in_specs=[
            pl.BlockSpec((block_rows, D), lambda i: (i, 0)),
            pl.BlockSpec((block_rows, D), lambda i, _o=off: (i + _o, 0)),
            pl.BlockSpec((D, D), lambda i: (0, 0)),
        ],
        out_specs=pl.BlockSpec((block_rows, D), lambda i: (i, 0)),
        out_shape=jax.ShapeDtypeStruct((n, D), jnp.float32),
    )(partials, partials, wh)


def _sc_body(sub_h, rel_h, obj_h, r_idx_h, q_rel_h, hx_h, rx_h, wqr_h, sp_h,
             out_h,
             subB, relB, objB, ridxB, qidx_v, obj_u, hsx0, hrx0, hsx1, hrx1,
             mq0, mq1, mm, alpha_v, spv, qrl, acc, gsem0, gsem1, qsem):
    cid = lax.axis_index("c")
    sid = lax.axis_index("s")
    wid = sid * 2 + cid
    ebase = wid * E_MAIN

    pltpu.sync_copy(sp_h, spv)
    pltpu.sync_copy(q_rel_h, qrl.at[pl.ds(0, B_Q)])
    b0 = spv[pl.ds(32, 16)]
    b1 = spv[pl.ds(48, 16)]
    wv0 = spv[pl.ds(0, 16)]
    wv1 = spv[pl.ds(16, 16)]
    wb = spv[pl.ds(64, 16)][0]

    z16 = jnp.zeros((16,), jnp.float32)

    def _zrow(r, carry):
        for k in range(D // 16):
            mm[r, pl.ds(k * 16, 16)] = z16
        return carry

    lax.fori_loop(0, C, _zrow, 0)
    z0 = sid * (N_NODE // 16)
    for j in range(19):
        pltpu.sync_copy(mm, acc.at[pl.ds(z0 + j * C, C), :])
    pltpu.sync_copy(mm.at[pl.ds(0, 17), :], acc.at[pl.ds(z0 + 19 * C, 17), :])
    plsc.subcore_barrier()

    lane = lax.iota(jnp.int32, 16)
    nm1 = jnp.full((16,), N_NODE - 1, jnp.int32)
    perms = [lane ^ k for k in (1, 2, 4, 8)]
    hbufs = (hsx0, hsx1)
    rbufs = (hrx0, hrx1)
    gsems = (gsem0, gsem1)
    mqs = (mq0, mq1)

    def _compose(kq):
        off = kq * C

        @plsc.parallel_loop(0, C // 16, 1, unroll=2)
        def _cg(g):
            r16 = ridxB[pl.ds(off + g * 16, 16)]
            qv = jnp.zeros((16,), jnp.int32)
            for j in range(16):
                qv = jnp.where(lane == j, qrl[pl.ds(r16[j], 16)][0], qv)
            qidx_v[pl.ds(g * 16, 16)] = qv

    def _fire_rows(kq, s):
        off = kq * C
        pltpu.async_copy(hx_h.at[subB.at[pl.ds(off, C)]], hbufs[s], gsems[s])
        pltpu.async_copy(rx_h.at[relB.at[pl.ds(off, C)]], rbufs[s], gsems[s])

    def _fire_wqr(kq, s):
        _compose(kq)
        pltpu.async_copy(wqr_h.at[qidx_v], mqs[s], qsem)

    def _wait_rows(s):
        pltpu.make_async_copy(hx_h.at[subB.at[pl.ds(0, C)]], hbufs[s],
                              gsems[s]).wait()
        pltpu.make_async_copy(rx_h.at[relB.at[pl.ds(0, C)]], rbufs[s],
                              gsems[s]).wait()

    def _wait_wqr(s):
        pltpu.make_async_copy(wqr_h.at[qidx_v], mqs[s], qsem).wait()

    def _process(kq, s, fire_next):
        hx = hbufs[s]
        rx = rbufs[s]
        mq = mqs[s]
        _wait_rows(s)
        _wait_wqr(s)
        if fire_next is not None:
            kn, cond = fire_next
            if cond is None:
                _fire_wqr(kn, 1 - s)
            else:
                @pl.when(cond)
                def _():
                    _fire_wqr(kn, 1 - s)
        off = kq * C
        for g in range(C // 16):
            obj_u[pl.ds(g * 16, 16)] = jnp.minimum(
                objB[pl.ds(off + g * 16, 16)], nm1)
        @plsc.parallel_loop(0, C, 1, unroll=4)
        def _edge(e):
            v0 = jnp.maximum(hx[e, pl.ds(D, 16)] + rx[e, pl.ds(D, 16)]
                             + mq[e, pl.ds(0, 16)] + b0, 0.0) * wv0
            v1 = jnp.maximum(hx[e, pl.ds(D + 16, 16)]
                             + rx[e, pl.ds(D + 16, 16)]
                             + mq[e, pl.ds(16, 16)] + b1, 0.0) * wv1
            s = v0 + v1
            for p in perms:
                s = s + s.at[p].get(mode="promise_in_bounds")
            a = 1.0 / (1.0 + jnp.exp(-(s + wb)))
            for kk in range(D // 16):
                sl = pl.ds(kk * 16, 16)
                mm[e, sl] = hx[e, sl] * rx[e, sl] * a

        pltpu.sync_copy(mm, acc.at[obj_u], add=True)

    def _quarter(q, carry):
        qbase = ebase + q * QE
        pltpu.sync_copy(sub_h.at[pl.ds(qbase, QE)], subB)
        pltpu.sync_copy(rel_h.at[pl.ds(qbase, QE)], relB)
        pltpu.sync_copy(obj_h.at[pl.ds(qbase, QE)], objB)
        pltpu.sync_copy(r_idx_h.at[pl.ds(qbase, QE)], ridxB)
        _fire_rows(0, 0)
        _fire_wqr(0, 0)

        def _pair(t, pc):
            k0 = t * 2
            _fire_rows(k0 + 1, 1)
            _process(k0, 0, (k0 + 1, None))

            @pl.when(t < QCH // 2 - 1)
            def _():
                _fire_rows(k0 + 2, 0)
            _process(k0 + 1, 1, (k0 + 2, t < QCH // 2 - 1))
            return pc

        lax.fori_loop(0, QCH // 2, _pair, 0)
        return carry

    lax.fori_loop(0, 12, _quarter, 0)

    @pl.when(wid < N_EPI)
    def _():
        base = NW * E_MAIN + wid * C
        pltpu.sync_copy(sub_h.at[pl.ds(base, C)], subB.at[pl.ds(0, C)])
        pltpu.sync_copy(rel_h.at[pl.ds(base, C)], relB.at[pl.ds(0, C)])
        pltpu.sync_copy(obj_h.at[pl.ds(base, C)], objB.at[pl.ds(0, C)])
        pltpu.sync_copy(r_idx_h.at[pl.ds(base, C)], ridxB.at[pl.ds(0, C)])
        _fire_rows(0, 0)
        _fire_wqr(0, 0)
        _process(0, 0, None)

    plsc.subcore_barrier()
    o0 = cid * N_NODE + sid * ROWS_A

    @pl.when(sid < 15)
    def _():
        pltpu.sync_copy(acc.at[pl.ds(sid * ROWS_A, ROWS_A), :],
                        out_h.at[pl.ds(o0, ROWS_A), :])

    @pl.when(sid == 15)
    def _():
        pltpu.sync_copy(acc.at[pl.ds(15 * ROWS_A, N_NODE - 15 * ROWS_A), :],
                        out_h.at[pl.ds(o0, N_NODE - 15 * ROWS_A), :])


@functools.cache
def _sc_edges_fn():
  return pl.kernel(
    _sc_body,
    out_type=jax.ShapeDtypeStruct((2 * N_NODE, D), jnp.float32),
    mesh=plsc.VectorSubcoreMesh(core_axis_name="c", subcore_axis_name="s",
                                num_cores=2, num_subcores=16),
    compiler_params=pltpu.CompilerParams(needs_layout_passes=False),
    scratch_types=[
        pltpu.VMEM((QE,), jnp.int32),
        pltpu.VMEM((QE,), jnp.int32),
        pltpu.VMEM((QE,), jnp.int32),
        pltpu.VMEM((QE,), jnp.int32),
        pltpu.VMEM((C,), jnp.int32),
        pltpu.VMEM((C,), jnp.int32),
        pltpu.VMEM((C, DX), jnp.float32),
        pltpu.VMEM((C, DX), jnp.float32),
        pltpu.VMEM((C, DX), jnp.float32),
        pltpu.VMEM((C, DX), jnp.float32),
        pltpu.VMEM((C, D), jnp.float32),
        pltpu.VMEM((C, D), jnp.float32),
        pltpu.VMEM((C, D), jnp.float32),
        pltpu.VMEM((C,), jnp.float32),
        pltpu.VMEM((80,), jnp.float32),
        pltpu.VMEM((B_Q + 16,), jnp.int32),
        pltpu.VMEM_SHARED((N_NODE, D), jnp.float32),
        pltpu.SemaphoreType.DMA,
        pltpu.SemaphoreType.DMA,
        pltpu.SemaphoreType.DMA,
    ],
  )


def kernel(q_sub, q_rel, r_idx, hidden, edges, n_node, rela_embed, Ws, Wr,
           Wqr_W, Wqr_b, walpha_W, walpha_b, Wh):
    sub_e = edges[:, 0].astype(jnp.int32)
    rel_e = edges[:, 1].astype(jnp.int32)
    obj_e = edges[:, 2].astype(jnp.int32)
    a_sub = _mm(hidden, Ws)
    a_rel = _mm(rela_embed, Wr)
    wqr_pre = _mm(rela_embed, Wqr_W)
    hx = jnp.concatenate(
        [hidden, a_sub, jnp.zeros((N_NODE, DX - D - A), jnp.float32)], axis=1)
    rx = jnp.concatenate(
        [rela_embed, a_rel,
         jnp.zeros((rela_embed.shape[0], DX - D - A), jnp.float32)], axis=1)
    wqrx = jnp.concatenate(
        [wqr_pre, jnp.zeros((wqr_pre.shape[0], D - A), jnp.float32)], axis=1)
    sp = jnp.concatenate([
        walpha_W.reshape(-1), Wqr_b.reshape(-1), walpha_b.reshape(-1),
        jnp.zeros((80 - A - A - 1,), jnp.float32),
    ])
    partials = _sc_edges_fn()(sub_e, rel_e, obj_e, r_idx.astype(jnp.int32),
                              q_rel.astype(jnp.int32), hx, rx, wqrx, sp)
    return _post(partials, Wh)[:N_NODE]
